# baseline (speedup 1.0000x reference)
import functools
import jax
import jax.numpy as jnp
from jax import lax
from jax.experimental import pallas as pl
from jax.experimental.pallas import tpu as pltpu
from jax.experimental.pallas import tpu_sc as plsc

NUM_C = 7
B, H, W = 16, 256, 256
N_WORKERS = 32
CHUNK_ROWS = 16
ROWS_PER_WORKER = (B * H) // N_WORKERS
CHUNKS_PER_WORKER = ROWS_PER_WORKER // CHUNK_ROWS
LANES = 16


def _sc_onehot_body(frame_hbm, out_hbm, in_buf, oh_buf):
    cid = lax.axis_index("c")
    sid = lax.axis_index("s")
    w = sid * 2 + cid
    b = w // 2
    r_base = (w % 2) * ROWS_PER_WORKER

    def chunk_body(k, carry):
        r0 = r_base + k * CHUNK_ROWS
        pltpu.sync_copy(frame_hbm.at[b, pl.ds(r0, CHUNK_ROWS)], in_buf)

        def row_body(i, carry2):
            def vec_body(kk, carry3):
                f = in_buf[i, pl.ds(kk * LANES, LANES)]
                for c in range(NUM_C):
                    oh_buf[c, i, pl.ds(kk * LANES, LANES)] = jnp.where(
                        f == c, jnp.float32(1.0), jnp.float32(0.0))
                return carry3
            return lax.fori_loop(0, W // LANES, vec_body, carry2)
        lax.fori_loop(0, CHUNK_ROWS, row_body, 0)

        for c in range(NUM_C):
            pltpu.sync_copy(oh_buf.at[c],
                            out_hbm.at[b, 3 * c + 2, pl.ds(r0, CHUNK_ROWS)])
        return carry
    lax.fori_loop(0, CHUNKS_PER_WORKER, chunk_body, 0)


def _sc_onehot(frame):
    mesh = plsc.VectorSubcoreMesh(
        core_axis_name="c", subcore_axis_name="s", num_cores=2, num_subcores=16)
    return pl.kernel(
        _sc_onehot_body,
        out_type=jax.ShapeDtypeStruct((B, 3 * NUM_C, H, W), jnp.float32),
        mesh=mesh,
        scratch_types=[
            pltpu.VMEM((CHUNK_ROWS, W), jnp.int32),
            pltpu.VMEM((NUM_C, CHUNK_ROWS, W), jnp.float32),
        ],
    )(frame)


def _tc_fill_body(_, out_ref):
    g = pl.program_id(1)
    rows = jax.lax.broadcasted_iota(jnp.int32, (H, W), 0).astype(jnp.float32)
    cols = jax.lax.broadcasted_iota(jnp.int32, (H, W), 1).astype(jnp.float32)
    out_ref[0, 0] = jnp.where(g < NUM_C, rows, cols)


def _tc_fill(buf):
    def idx_map(bb, g):
        ch = jnp.where(g < NUM_C, 3 * g, 3 * (g - NUM_C) + 1)
        return (bb, ch, 0, 0)
    return pl.pallas_call(
        _tc_fill_body,
        grid=(B, 2 * NUM_C),
        in_specs=[pl.BlockSpec(memory_space=pl.ANY)],
        out_specs=pl.BlockSpec((1, 1, H, W), idx_map),
        out_shape=jax.ShapeDtypeStruct((B, 3 * NUM_C, H, W), jnp.float32),
        input_output_aliases={0: 0},
    )(buf)


def kernel(frame, embed_weights):
    del embed_weights
    return _tc_fill(_sc_onehot(frame))


# speedup vs baseline: 3.7497x; 3.7497x over previous
import jax
import jax.numpy as jnp
from jax.experimental import pallas as pl

NUM_C = 7
ROW_BLK = 128


def _onehot_kernel(frame_ref, out_ref):
    r = pl.program_id(1)
    f = frame_ref[0]
    rows = jax.lax.broadcasted_iota(jnp.int32, (ROW_BLK, 256), 0)
    loc_x = (rows + r * ROW_BLK).astype(jnp.float32)
    loc_y = jax.lax.broadcasted_iota(jnp.int32, (ROW_BLK, 256), 1).astype(jnp.float32)
    for c in range(NUM_C):
        out_ref[0, 3 * c] = loc_x
        out_ref[0, 3 * c + 1] = loc_y
        out_ref[0, 3 * c + 2] = (f == c).astype(jnp.float32)


def kernel(frame, embed_weights):
    del embed_weights
    B, H, W = frame.shape
    grid = (B, H // ROW_BLK)
    return pl.pallas_call(
        _onehot_kernel,
        grid=grid,
        in_specs=[pl.BlockSpec((1, ROW_BLK, W), lambda b, r: (b, r, 0))],
        out_specs=pl.BlockSpec((1, 3 * NUM_C, ROW_BLK, W), lambda b, r: (b, 0, r, 0)),
        out_shape=jax.ShapeDtypeStruct((B, 3 * NUM_C, H, W), jnp.float32),
    )(frame)


# speedup vs baseline: 4.6851x; 1.2495x over previous
import jax
import jax.numpy as jnp
from jax.experimental import pallas as pl

NUM_C = 7
ROW_BLK = 256


def _onehot_kernel(frame_ref, out_ref):
    r = pl.program_id(1)
    f = frame_ref[0]
    rows = jax.lax.broadcasted_iota(jnp.int32, (ROW_BLK, 256), 0)
    loc_x = (rows + r * ROW_BLK).astype(jnp.float32)
    loc_y = jax.lax.broadcasted_iota(jnp.int32, (ROW_BLK, 256), 1).astype(jnp.float32)
    for c in range(NUM_C):
        out_ref[0, 3 * c] = loc_x
        out_ref[0, 3 * c + 1] = loc_y
        out_ref[0, 3 * c + 2] = (f == c).astype(jnp.float32)


def kernel(frame, embed_weights):
    del embed_weights
    B, H, W = frame.shape
    grid = (B, H // ROW_BLK)
    return pl.pallas_call(
        _onehot_kernel,
        grid=grid,
        in_specs=[pl.BlockSpec((1, ROW_BLK, W), lambda b, r: (b, r, 0))],
        out_specs=pl.BlockSpec((1, 3 * NUM_C, ROW_BLK, W), lambda b, r: (b, 0, r, 0)),
        out_shape=jax.ShapeDtypeStruct((B, 3 * NUM_C, H, W), jnp.float32),
    )(frame)
